# baseline (speedup 1.0000x reference)
import functools

import jax
import jax.numpy as jnp
from jax import lax
from jax.experimental import pallas as pl
from jax.experimental.pallas import tpu as pltpu
from jax.experimental.pallas import tpu_sc as plsc

EMBED = 32
BATCH = 16384

NC = 2
NS = 16
LANES = 16
NW = NC * NS
BPW = BATCH // NW
GCHUNK = 128
NCHUNK = BPW // GCHUNK


def _sc_body(user, item, user_table, item_table, out,
             idx_u, idx_i, rows_u, rows_i, out_v, sem):
    wid = lax.axis_index("s") * NC + lax.axis_index("c")
    base = pl.multiple_of(wid * BPW, BPW)

    pltpu.sync_copy(user.at[pl.ds(base, BPW)], idx_u)
    pltpu.sync_copy(item.at[pl.ds(base, BPW)], idx_i)

    copies = []
    for c in range(NCHUNK):
        sl = pl.ds(c * GCHUNK, GCHUNK)
        copies.append(pltpu.make_async_copy(
            user_table.at[idx_u.at[sl]], rows_u.at[sl], sem))
        copies.append(pltpu.make_async_copy(
            item_table.at[idx_i.at[sl]], rows_i.at[sl], sem))
    for cp in copies:
        cp.start()
    for cp in copies:
        cp.wait()

    def group(g, carry):
        row_ids = g * LANES + lax.iota(jnp.int32, LANES)
        acc = jnp.zeros((LANES,), jnp.float32)
        for e in range(EMBED):
            col = jnp.full((LANES,), e, jnp.int32)
            uu = plsc.load_gather(rows_u, [row_ids, col])
            ii = plsc.load_gather(rows_i, [row_ids, col])
            acc = acc + uu * ii
        y = 1.0 / (1.0 + jnp.exp(-acc))
        out_v[pl.ds(pl.multiple_of(g * LANES, LANES), LANES)] = y
        return carry

    lax.fori_loop(0, BPW // LANES, group, 0)

    pltpu.sync_copy(out_v, out.at[pl.ds(base, BPW)])


@jax.jit
def _pmf_forward(user, item, user_table, item_table):
    mesh = plsc.VectorSubcoreMesh(core_axis_name="c", subcore_axis_name="s")
    kern = functools.partial(
        pl.kernel,
        out_type=jax.ShapeDtypeStruct((BATCH,), jnp.float32),
        mesh=mesh,
        scratch_types=[
            pltpu.VMEM((BPW,), jnp.int32),
            pltpu.VMEM((BPW,), jnp.int32),
            pltpu.VMEM((BPW, EMBED), jnp.float32),
            pltpu.VMEM((BPW, EMBED), jnp.float32),
            pltpu.VMEM((BPW,), jnp.float32),
            pltpu.SemaphoreType.DMA,
        ],
        compiler_params=pltpu.CompilerParams(
            needs_layout_passes=False, use_tc_tiling_on_sc=False),
    )(_sc_body)
    return kern(user, item, user_table, item_table)


def kernel(user, item, user_table, item_table):
    return _pmf_forward(user, item, user_table, item_table)
